# baseline (speedup 1.0000x reference)
import functools

import jax
import jax.numpy as jnp
from jax.experimental import pallas as pl
from jax.experimental.pallas import tpu as pltpu
from jax.experimental.pallas import tpu_sc as plsc

V, D, H2, L, T = 30000, 256, 512, 2, 4
H = H2 // 2
B, S = 32, 512
G4 = 4 * H
M = B * S

_WIN = 128


def _emb_gather(table, ids_flat):
    mesh = plsc.VectorSubcoreMesh(core_axis_name="core",
                                  subcore_axis_name="subcore")
    idx2 = ids_flat.reshape(1, M)

    @functools.partial(
        pl.kernel,
        out_type=jax.ShapeDtypeStruct((M, D), jnp.float32),
        mesh=mesh,
    )
    def k(tab_hbm, i_hbm, o_hbm):
        def body(i_vmem, o_vmem):
            pltpu.sync_copy(tab_hbm.at[i_vmem.at[0]], o_vmem)

        pltpu.emit_pipeline(
            body,
            grid=(M // _WIN,),
            in_specs=[pl.BlockSpec((1, _WIN), index_map=lambda i: (0, i))],
            out_specs=[pl.BlockSpec((_WIN, D), index_map=lambda i: (i, 0))],
            core_axis_name=("core", "subcore"),
            dimension_semantics=(pltpu.PARALLEL,),
        )(i_hbm, o_hbm)

    return k(table, idx2)


_CHUNK = 32
_NBLK = S // _CHUNK


def _lstm_step(x_gates, h, c, w):
    g = x_gates.astype(jnp.float32) + jnp.dot(
        h.astype(jnp.bfloat16), w, preferred_element_type=jnp.float32)
    sif = 0.5 * jnp.tanh(g[:, : 2 * H] * 0.5) + 0.5
    gg = jnp.tanh(g[:, 2 * H: 3 * H])
    so = 0.5 * jnp.tanh(g[:, 3 * H:] * 0.5) + 0.5
    c2 = sif[:, H:] * c + sif[:, :H] * gg
    h2 = so * jnp.tanh(c2)
    return h2, c2


def _zero_state(hf_ref, cf_ref, hb_ref, cb_ref):
    @pl.when(pl.program_id(0) == 0)
    def _():
        z = jnp.zeros((B, H), jnp.float32)
        hf_ref[...] = z
        cf_ref[...] = z
        hb_ref[...] = z
        cb_ref[...] = z


def _store_h(ref, j, h):
    if len(ref.shape) == 3:
        ref[j] = h.astype(jnp.bfloat16)
    else:
        ref[pl.ds(j * B, B)] = h.astype(jnp.bfloat16)


def _run_steps(xpf_ref, xpb_ref, wf_ref, wb_ref, of_ref, ob_ref,
               hf_ref, cf_ref, hb_ref, cb_ref):
    wf = wf_ref[...]
    wb = wb_ref[...]
    for j in range(_CHUNK):
        h2, c2 = _lstm_step(xpf_ref[pl.ds(j * B, B)], hf_ref[...],
                            cf_ref[...], wf)
        hf_ref[...] = h2
        cf_ref[...] = c2
        _store_h(of_ref, j, h2)
        jb = _CHUNK - 1 - j
        h2, c2 = _lstm_step(xpb_ref[pl.ds(jb * B, B)], hb_ref[...],
                            cb_ref[...], wb)
        hb_ref[...] = h2
        cb_ref[...] = c2
        _store_h(ob_ref, jb, h2)


def _prep_body(wi00_ref, wi01_ref, wi10_ref, wi11_ref,
               wh00_ref, wh01_ref, wh10_ref, wh11_ref,
               bi00_ref, bh00_ref, bi01_ref, bh01_ref,
               bi10_ref, bh10_ref, bi11_ref, bh11_ref, f2_ref,
               w0_ref, w1a_ref, w1b_ref, wf0_ref, wb0_ref, wf1_ref, wb1_ref,
               b0_ref, b1_ref, fa_ref, fb_ref):
    bf = jnp.bfloat16
    w0_ref[:, :G4] = wi00_ref[...].T.astype(bf)
    w0_ref[:, G4:] = wi01_ref[...].T.astype(bf)
    w1a_ref[:, :G4] = wi10_ref[:, :H].T.astype(bf)
    w1a_ref[:, G4:] = wi11_ref[:, :H].T.astype(bf)
    w1b_ref[:, :G4] = wi10_ref[:, H:].T.astype(bf)
    w1b_ref[:, G4:] = wi11_ref[:, H:].T.astype(bf)
    wf0_ref[...] = wh00_ref[...].T.astype(bf)
    wb0_ref[...] = wh01_ref[...].T.astype(bf)
    wf1_ref[...] = wh10_ref[...].T.astype(bf)
    wb1_ref[...] = wh11_ref[...].T.astype(bf)
    b0_ref[:, :G4] = bi00_ref[...] + bh00_ref[...]
    b0_ref[:, G4:] = bi01_ref[...] + bh01_ref[...]
    b1_ref[:, :G4] = bi10_ref[...] + bh10_ref[...]
    b1_ref[:, G4:] = bi11_ref[...] + bh11_ref[...]
    fa_ref[...] = jnp.zeros((H, 32), bf)
    fb_ref[...] = jnp.zeros((H, 32), bf)
    fa_ref[:, : T * 5] = f2_ref[:, :H].T.astype(bf)
    fb_ref[:, : T * 5] = f2_ref[:, H:].T.astype(bf)


def _prep_weights(wi00, wi01, wi10, wi11, wh00, wh01, wh10, wh11,
                  biases, f2):
    ins = [wi00, wi01, wi10, wi11, wh00, wh01, wh10, wh11] + biases + [f2]
    outs = [
        jax.ShapeDtypeStruct((D, 2 * G4), jnp.bfloat16),
        jax.ShapeDtypeStruct((H, 2 * G4), jnp.bfloat16),
        jax.ShapeDtypeStruct((H, 2 * G4), jnp.bfloat16),
        jax.ShapeDtypeStruct((H, G4), jnp.bfloat16),
        jax.ShapeDtypeStruct((H, G4), jnp.bfloat16),
        jax.ShapeDtypeStruct((H, G4), jnp.bfloat16),
        jax.ShapeDtypeStruct((H, G4), jnp.bfloat16),
        jax.ShapeDtypeStruct((1, 2 * G4), jnp.float32),
        jax.ShapeDtypeStruct((1, 2 * G4), jnp.float32),
        jax.ShapeDtypeStruct((H, 32), jnp.bfloat16),
        jax.ShapeDtypeStruct((H, 32), jnp.bfloat16),
    ]
    return pl.pallas_call(_prep_body, out_shape=outs)(*ins)


def _fscan1_body(xa_ref, xd_ref, win_ref, b_ref, wf_ref, wb_ref,
                 of_ref, ob_ref, xpf_ref, xpb_ref,
                 hf_ref, cf_ref, hb_ref, cb_ref):
    _zero_state(hf_ref, cf_ref, hb_ref, cb_ref)
    xa = xa_ref[...].reshape(_CHUNK * B, D).astype(jnp.bfloat16)
    xd = xd_ref[...].reshape(_CHUNK * B, D).astype(jnp.bfloat16)
    xpf_ref[...] = jnp.dot(xa, win_ref[:, :G4],
                           preferred_element_type=jnp.float32) + b_ref[:, :G4]
    xpb_ref[...] = jnp.dot(xd, win_ref[:, G4:],
                           preferred_element_type=jnp.float32) + b_ref[:, G4:]
    _run_steps(xpf_ref, xpb_ref, wf_ref, wb_ref, of_ref, ob_ref,
               hf_ref, cf_ref, hb_ref, cb_ref)


def _fused_scan_l0(x, win, b, wfT, wbT):
    return pl.pallas_call(
        _fscan1_body,
        grid=(_NBLK,),
        in_specs=[
            pl.BlockSpec((_CHUNK, B, D), lambda i: (i, 0, 0)),
            pl.BlockSpec((_CHUNK, B, D), lambda i: (_NBLK - 1 - i, 0, 0)),
            pl.BlockSpec((D, 2 * G4), lambda i: (0, 0)),
            pl.BlockSpec((1, 2 * G4), lambda i: (0, 0)),
            pl.BlockSpec((H, G4), lambda i: (0, 0)),
            pl.BlockSpec((H, G4), lambda i: (0, 0)),
        ],
        out_specs=[
            pl.BlockSpec((_CHUNK, B, H), lambda i: (i, 0, 0)),
            pl.BlockSpec((_CHUNK, B, H), lambda i: (_NBLK - 1 - i, 0, 0)),
        ],
        out_shape=[
            jax.ShapeDtypeStruct((S, B, H), jnp.bfloat16),
            jax.ShapeDtypeStruct((S, B, H), jnp.bfloat16),
        ],
        scratch_shapes=[
            pltpu.VMEM((_CHUNK * B, G4), jnp.float32),
            pltpu.VMEM((_CHUNK * B, G4), jnp.float32),
            pltpu.VMEM((B, H), jnp.float32),
            pltpu.VMEM((B, H), jnp.float32),
            pltpu.VMEM((B, H), jnp.float32),
            pltpu.VMEM((B, H), jnp.float32),
        ],
        compiler_params=pltpu.CompilerParams(
            dimension_semantics=("arbitrary",)),
    )(x, x, win, b.reshape(1, 2 * G4), wfT, wbT)


def _fscan2_body(ha_ref, hb0a_ref, hd_ref, hb0d_ref, wina_ref, winb_ref,
                 b_ref, wf_ref, wb_ref, fa_ref, fb_ref,
                 yf_ref, yb_ref, xpf_ref, xpb_ref, osf_ref, osb_ref,
                 hf_ref, cf_ref, hb_ref, cb_ref):
    _zero_state(hf_ref, cf_ref, hb_ref, cb_ref)
    ha = ha_ref[...].reshape(_CHUNK * B, H)
    h0a = hb0a_ref[...].reshape(_CHUNK * B, H)
    hd = hd_ref[...].reshape(_CHUNK * B, H)
    h0d = hb0d_ref[...].reshape(_CHUNK * B, H)
    xpf_ref[...] = (
        jnp.dot(ha, wina_ref[:, :G4], preferred_element_type=jnp.float32)
        + jnp.dot(h0a, winb_ref[:, :G4], preferred_element_type=jnp.float32)
        + b_ref[:, :G4])
    xpb_ref[...] = (
        jnp.dot(hd, wina_ref[:, G4:], preferred_element_type=jnp.float32)
        + jnp.dot(h0d, winb_ref[:, G4:], preferred_element_type=jnp.float32)
        + b_ref[:, G4:])
    _run_steps(xpf_ref, xpb_ref, wf_ref, wb_ref, osf_ref, osb_ref,
               hf_ref, cf_ref, hb_ref, cb_ref)
    yf_ref[...] = jnp.dot(
        osf_ref[...], fa_ref[...],
        preferred_element_type=jnp.float32).reshape(_CHUNK, B, 32)
    yb_ref[...] = jnp.dot(
        osb_ref[...], fb_ref[...],
        preferred_element_type=jnp.float32).reshape(_CHUNK, B, 32)


def _fused_scan_l1(hf0, hb0, wina, winb, b, wfT, wbT, fa, fb):
    return pl.pallas_call(
        _fscan2_body,
        grid=(_NBLK,),
        in_specs=[
            pl.BlockSpec((_CHUNK, B, H), lambda i: (i, 0, 0)),
            pl.BlockSpec((_CHUNK, B, H), lambda i: (i, 0, 0)),
            pl.BlockSpec((_CHUNK, B, H), lambda i: (_NBLK - 1 - i, 0, 0)),
            pl.BlockSpec((_CHUNK, B, H), lambda i: (_NBLK - 1 - i, 0, 0)),
            pl.BlockSpec((H, 2 * G4), lambda i: (0, 0)),
            pl.BlockSpec((H, 2 * G4), lambda i: (0, 0)),
            pl.BlockSpec((1, 2 * G4), lambda i: (0, 0)),
            pl.BlockSpec((H, G4), lambda i: (0, 0)),
            pl.BlockSpec((H, G4), lambda i: (0, 0)),
            pl.BlockSpec((H, 32), lambda i: (0, 0)),
            pl.BlockSpec((H, 32), lambda i: (0, 0)),
        ],
        out_specs=[
            pl.BlockSpec((_CHUNK, B, 32), lambda i: (i, 0, 0)),
            pl.BlockSpec((_CHUNK, B, 32), lambda i: (_NBLK - 1 - i, 0, 0)),
        ],
        out_shape=[
            jax.ShapeDtypeStruct((S, B, 32), jnp.float32),
            jax.ShapeDtypeStruct((S, B, 32), jnp.float32),
        ],
        scratch_shapes=[
            pltpu.VMEM((_CHUNK * B, G4), jnp.float32),
            pltpu.VMEM((_CHUNK * B, G4), jnp.float32),
            pltpu.VMEM((_CHUNK * B, H), jnp.bfloat16),
            pltpu.VMEM((_CHUNK * B, H), jnp.bfloat16),
            pltpu.VMEM((B, H), jnp.float32),
            pltpu.VMEM((B, H), jnp.float32),
            pltpu.VMEM((B, H), jnp.float32),
            pltpu.VMEM((B, H), jnp.float32),
        ],
        compiler_params=pltpu.CompilerParams(
            dimension_semantics=("arbitrary",)),
    )(hf0, hb0, hf0, hb0, wina, winb, b.reshape(1, 2 * G4), wfT, wbT, fa, fb)


def kernel(input_ids, emb_table, Wih_l0_d0, Whh_l0_d0, bih_l0_d0, bhh_l0_d0,
           Wih_l0_d1, Whh_l0_d1, bih_l0_d1, bhh_l0_d1, Wih_l1_d0, Whh_l1_d0,
           bih_l1_d0, bhh_l1_d0, Wih_l1_d1, Whh_l1_d1, bih_l1_d1, bhh_l1_d1,
           fc_w, fc_b):
    biases = [b.reshape(1, G4) for b in
              (bih_l0_d0, bhh_l0_d0, bih_l0_d1, bhh_l0_d1,
               bih_l1_d0, bhh_l1_d0, bih_l1_d1, bhh_l1_d1)]
    (w0, w1a, w1b, wf0, wb0, wf1, wb1, b0, b1, fa, fb) = _prep_weights(
        Wih_l0_d0, Wih_l0_d1, Wih_l1_d0, Wih_l1_d1,
        Whh_l0_d0, Whh_l0_d1, Whh_l1_d0, Whh_l1_d1,
        biases, fc_w.reshape(T * 5, H2))

    ids_tm = input_ids.T.reshape(M).astype(jnp.int32)
    x = _emb_gather(emb_table, ids_tm)

    hf0, hb0 = _fused_scan_l0(x.reshape(S, B, D), w0, b0, wf0, wb0)

    yf, yb = _fused_scan_l1(hf0, hb0, w1a, w1b, b1, wf1, wb1, fa, fb)
    y = yf[:, :, : T * 5] + yb[:, :, : T * 5] + fc_b.reshape(T * 5)
    logits = y.reshape(S, B, T, 5).transpose(1, 2, 0, 3)
    return logits


# speedup vs baseline: 1.0213x; 1.0213x over previous
import functools

import jax
import jax.numpy as jnp
from jax.experimental import pallas as pl
from jax.experimental.pallas import tpu as pltpu
from jax.experimental.pallas import tpu_sc as plsc

V, D, H2, L, T = 30000, 256, 512, 2, 4
H = H2 // 2
B, S = 32, 512
G4 = 4 * H
M = B * S

_WIN = 128


def _emb_gather(table, ids_flat):
    mesh = plsc.VectorSubcoreMesh(core_axis_name="core",
                                  subcore_axis_name="subcore")
    idx2 = ids_flat.reshape(1, M)

    @functools.partial(
        pl.kernel,
        out_type=jax.ShapeDtypeStruct((M, D), jnp.float32),
        mesh=mesh,
    )
    def k(tab_hbm, i_hbm, o_hbm):
        def body(i_vmem, o_vmem):
            pltpu.sync_copy(tab_hbm.at[i_vmem.at[0]], o_vmem)

        pltpu.emit_pipeline(
            body,
            grid=(M // _WIN,),
            in_specs=[pl.BlockSpec((1, _WIN), index_map=lambda i: (0, i))],
            out_specs=[pl.BlockSpec((_WIN, D), index_map=lambda i: (i, 0))],
            core_axis_name=("core", "subcore"),
            dimension_semantics=(pltpu.PARALLEL,),
        )(i_hbm, o_hbm)

    return k(table, idx2)


_CHUNK = 32
_NBLK = S // _CHUNK


def _lstm_step(x_gates, h, c, w):
    g = x_gates.astype(jnp.float32) + jnp.dot(
        h.astype(jnp.bfloat16), w, preferred_element_type=jnp.float32)
    sif = jax.nn.sigmoid(g[:, : 2 * H])
    gg = jnp.tanh(g[:, 2 * H: 3 * H])
    so = jax.nn.sigmoid(g[:, 3 * H:])
    c2 = sif[:, H:] * c + sif[:, :H] * gg
    h2 = so * jnp.tanh(c2)
    return h2, c2


def _zero_state(hf_ref, cf_ref, hb_ref, cb_ref):
    @pl.when(pl.program_id(0) == 0)
    def _():
        z = jnp.zeros((B, H), jnp.float32)
        hf_ref[...] = z
        cf_ref[...] = z
        hb_ref[...] = z
        cb_ref[...] = z


def _store_h(ref, j, h):
    if len(ref.shape) == 3:
        ref[j] = h.astype(jnp.bfloat16)
    else:
        ref[pl.ds(j * B, B)] = h.astype(jnp.bfloat16)


_NP = 4
_PP = _CHUNK // _NP


def _run_steps(projf, projb, xpf_refs, xpb_refs, wf_ref, wb_ref,
               of_ref, ob_ref, hf_ref, cf_ref, hb_ref, cb_ref):
    wf = wf_ref[...]
    wb = wb_ref[...]
    projf(0)
    projb(_NP - 1)
    for k in range(_NP):
        if k + 1 < _NP:
            projf(k + 1)
            projb(_NP - 2 - k)
        pb = _NP - 1 - k
        for j in range(k * _PP, (k + 1) * _PP):
            h2, c2 = _lstm_step(xpf_refs[k][pl.ds((j - k * _PP) * B, B)],
                                hf_ref[...], cf_ref[...], wf)
            hf_ref[...] = h2
            cf_ref[...] = c2
            _store_h(of_ref, j, h2)
            jb = _CHUNK - 1 - j
            h2, c2 = _lstm_step(xpb_refs[pb][pl.ds((jb - pb * _PP) * B, B)],
                                hb_ref[...], cb_ref[...], wb)
            hb_ref[...] = h2
            cb_ref[...] = c2
            _store_h(ob_ref, jb, h2)


def _prep_body(wi00_ref, wi01_ref, wi10_ref, wi11_ref,
               wh00_ref, wh01_ref, wh10_ref, wh11_ref,
               bi00_ref, bh00_ref, bi01_ref, bh01_ref,
               bi10_ref, bh10_ref, bi11_ref, bh11_ref, f2_ref,
               w0_ref, w1a_ref, w1b_ref, wf0_ref, wb0_ref, wf1_ref, wb1_ref,
               b0_ref, b1_ref, fa_ref, fb_ref):
    bf = jnp.bfloat16
    w0_ref[:, :G4] = wi00_ref[...].T.astype(bf)
    w0_ref[:, G4:] = wi01_ref[...].T.astype(bf)
    w1a_ref[:, :G4] = wi10_ref[:, :H].T.astype(bf)
    w1a_ref[:, G4:] = wi11_ref[:, :H].T.astype(bf)
    w1b_ref[:, :G4] = wi10_ref[:, H:].T.astype(bf)
    w1b_ref[:, G4:] = wi11_ref[:, H:].T.astype(bf)
    wf0_ref[...] = wh00_ref[...].T.astype(bf)
    wb0_ref[...] = wh01_ref[...].T.astype(bf)
    wf1_ref[...] = wh10_ref[...].T.astype(bf)
    wb1_ref[...] = wh11_ref[...].T.astype(bf)
    b0_ref[:, :G4] = bi00_ref[...] + bh00_ref[...]
    b0_ref[:, G4:] = bi01_ref[...] + bh01_ref[...]
    b1_ref[:, :G4] = bi10_ref[...] + bh10_ref[...]
    b1_ref[:, G4:] = bi11_ref[...] + bh11_ref[...]
    fa_ref[...] = jnp.zeros((H, 32), bf)
    fb_ref[...] = jnp.zeros((H, 32), bf)
    fa_ref[:, : T * 5] = f2_ref[:, :H].T.astype(bf)
    fb_ref[:, : T * 5] = f2_ref[:, H:].T.astype(bf)


def _prep_weights(wi00, wi01, wi10, wi11, wh00, wh01, wh10, wh11,
                  biases, f2):
    ins = [wi00, wi01, wi10, wi11, wh00, wh01, wh10, wh11] + biases + [f2]
    outs = [
        jax.ShapeDtypeStruct((D, 2 * G4), jnp.bfloat16),
        jax.ShapeDtypeStruct((H, 2 * G4), jnp.bfloat16),
        jax.ShapeDtypeStruct((H, 2 * G4), jnp.bfloat16),
        jax.ShapeDtypeStruct((H, G4), jnp.bfloat16),
        jax.ShapeDtypeStruct((H, G4), jnp.bfloat16),
        jax.ShapeDtypeStruct((H, G4), jnp.bfloat16),
        jax.ShapeDtypeStruct((H, G4), jnp.bfloat16),
        jax.ShapeDtypeStruct((1, 2 * G4), jnp.float32),
        jax.ShapeDtypeStruct((1, 2 * G4), jnp.float32),
        jax.ShapeDtypeStruct((H, 32), jnp.bfloat16),
        jax.ShapeDtypeStruct((H, 32), jnp.bfloat16),
    ]
    return pl.pallas_call(_prep_body, out_shape=outs)(*ins)


def _fscan1_body(xa_ref, xd_ref, win_ref, b_ref, wf_ref, wb_ref,
                 of_ref, ob_ref, *scratch):
    xpf_refs, xpb_refs = scratch[:_NP], scratch[_NP:2 * _NP]
    hf_ref, cf_ref, hb_ref, cb_ref = scratch[2 * _NP:]
    _zero_state(hf_ref, cf_ref, hb_ref, cb_ref)

    def projf(p):
        xa = xa_ref[pl.ds(p * _PP, _PP)].reshape(
            _PP * B, D).astype(jnp.bfloat16)
        xpf_refs[p][...] = jnp.dot(
            xa, win_ref[:, :G4],
            preferred_element_type=jnp.float32) + b_ref[:, :G4]

    def projb(p):
        xd = xd_ref[pl.ds(p * _PP, _PP)].reshape(
            _PP * B, D).astype(jnp.bfloat16)
        xpb_refs[p][...] = jnp.dot(
            xd, win_ref[:, G4:],
            preferred_element_type=jnp.float32) + b_ref[:, G4:]

    _run_steps(projf, projb, xpf_refs, xpb_refs, wf_ref, wb_ref,
               of_ref, ob_ref, hf_ref, cf_ref, hb_ref, cb_ref)


def _fused_scan_l0(x, win, b, wfT, wbT):
    return pl.pallas_call(
        _fscan1_body,
        grid=(_NBLK,),
        in_specs=[
            pl.BlockSpec((_CHUNK, B, D), lambda i: (i, 0, 0)),
            pl.BlockSpec((_CHUNK, B, D), lambda i: (_NBLK - 1 - i, 0, 0)),
            pl.BlockSpec((D, 2 * G4), lambda i: (0, 0)),
            pl.BlockSpec((1, 2 * G4), lambda i: (0, 0)),
            pl.BlockSpec((H, G4), lambda i: (0, 0)),
            pl.BlockSpec((H, G4), lambda i: (0, 0)),
        ],
        out_specs=[
            pl.BlockSpec((_CHUNK, B, H), lambda i: (i, 0, 0)),
            pl.BlockSpec((_CHUNK, B, H), lambda i: (_NBLK - 1 - i, 0, 0)),
        ],
        out_shape=[
            jax.ShapeDtypeStruct((S, B, H), jnp.bfloat16),
            jax.ShapeDtypeStruct((S, B, H), jnp.bfloat16),
        ],
        scratch_shapes=(
            [pltpu.VMEM((_PP * B, G4), jnp.float32) for _ in range(2 * _NP)]
            + [pltpu.VMEM((B, H), jnp.float32) for _ in range(4)]),
        compiler_params=pltpu.CompilerParams(
            dimension_semantics=("arbitrary",)),
    )(x, x, win, b.reshape(1, 2 * G4), wfT, wbT)


def _fscan2_body(ha_ref, hb0a_ref, hd_ref, hb0d_ref, wina_ref, winb_ref,
                 b_ref, wf_ref, wb_ref, fa_ref, fb_ref,
                 yf_ref, yb_ref, *scratch):
    xpf_refs, xpb_refs = scratch[:_NP], scratch[_NP:2 * _NP]
    osf_ref, osb_ref = scratch[2 * _NP:2 * _NP + 2]
    hf_ref, cf_ref, hb_ref, cb_ref = scratch[2 * _NP + 2:]
    _zero_state(hf_ref, cf_ref, hb_ref, cb_ref)

    def projf(p):
        ha = ha_ref[pl.ds(p * _PP, _PP)].reshape(_PP * B, H)
        h0a = hb0a_ref[pl.ds(p * _PP, _PP)].reshape(_PP * B, H)
        xpf_refs[p][...] = (
            jnp.dot(ha, wina_ref[:, :G4], preferred_element_type=jnp.float32)
            + jnp.dot(h0a, winb_ref[:, :G4],
                      preferred_element_type=jnp.float32)
            + b_ref[:, :G4])

    def projb(p):
        hd = hd_ref[pl.ds(p * _PP, _PP)].reshape(_PP * B, H)
        h0d = hb0d_ref[pl.ds(p * _PP, _PP)].reshape(_PP * B, H)
        xpb_refs[p][...] = (
            jnp.dot(hd, wina_ref[:, G4:], preferred_element_type=jnp.float32)
            + jnp.dot(h0d, winb_ref[:, G4:],
                      preferred_element_type=jnp.float32)
            + b_ref[:, G4:])

    _run_steps(projf, projb, xpf_refs, xpb_refs, wf_ref, wb_ref,
               osf_ref, osb_ref, hf_ref, cf_ref, hb_ref, cb_ref)
    yf_ref[...] = jnp.dot(
        osf_ref[...], fa_ref[...],
        preferred_element_type=jnp.float32).reshape(_CHUNK, B, 32)
    yb_ref[...] = jnp.dot(
        osb_ref[...], fb_ref[...],
        preferred_element_type=jnp.float32).reshape(_CHUNK, B, 32)


def _fused_scan_l1(hf0, hb0, wina, winb, b, wfT, wbT, fa, fb):
    return pl.pallas_call(
        _fscan2_body,
        grid=(_NBLK,),
        in_specs=[
            pl.BlockSpec((_CHUNK, B, H), lambda i: (i, 0, 0)),
            pl.BlockSpec((_CHUNK, B, H), lambda i: (i, 0, 0)),
            pl.BlockSpec((_CHUNK, B, H), lambda i: (_NBLK - 1 - i, 0, 0)),
            pl.BlockSpec((_CHUNK, B, H), lambda i: (_NBLK - 1 - i, 0, 0)),
            pl.BlockSpec((H, 2 * G4), lambda i: (0, 0)),
            pl.BlockSpec((H, 2 * G4), lambda i: (0, 0)),
            pl.BlockSpec((1, 2 * G4), lambda i: (0, 0)),
            pl.BlockSpec((H, G4), lambda i: (0, 0)),
            pl.BlockSpec((H, G4), lambda i: (0, 0)),
            pl.BlockSpec((H, 32), lambda i: (0, 0)),
            pl.BlockSpec((H, 32), lambda i: (0, 0)),
        ],
        out_specs=[
            pl.BlockSpec((_CHUNK, B, 32), lambda i: (i, 0, 0)),
            pl.BlockSpec((_CHUNK, B, 32), lambda i: (_NBLK - 1 - i, 0, 0)),
        ],
        out_shape=[
            jax.ShapeDtypeStruct((S, B, 32), jnp.float32),
            jax.ShapeDtypeStruct((S, B, 32), jnp.float32),
        ],
        scratch_shapes=(
            [pltpu.VMEM((_PP * B, G4), jnp.float32) for _ in range(2 * _NP)]
            + [pltpu.VMEM((_CHUNK * B, H), jnp.bfloat16) for _ in range(2)]
            + [pltpu.VMEM((B, H), jnp.float32) for _ in range(4)]),
        compiler_params=pltpu.CompilerParams(
            dimension_semantics=("arbitrary",)),
    )(hf0, hb0, hf0, hb0, wina, winb, b.reshape(1, 2 * G4), wfT, wbT, fa, fb)


def kernel(input_ids, emb_table, Wih_l0_d0, Whh_l0_d0, bih_l0_d0, bhh_l0_d0,
           Wih_l0_d1, Whh_l0_d1, bih_l0_d1, bhh_l0_d1, Wih_l1_d0, Whh_l1_d0,
           bih_l1_d0, bhh_l1_d0, Wih_l1_d1, Whh_l1_d1, bih_l1_d1, bhh_l1_d1,
           fc_w, fc_b):
    biases = [b.reshape(1, G4) for b in
              (bih_l0_d0, bhh_l0_d0, bih_l0_d1, bhh_l0_d1,
               bih_l1_d0, bhh_l1_d0, bih_l1_d1, bhh_l1_d1)]
    (w0, w1a, w1b, wf0, wb0, wf1, wb1, b0, b1, fa, fb) = _prep_weights(
        Wih_l0_d0, Wih_l0_d1, Wih_l1_d0, Wih_l1_d1,
        Whh_l0_d0, Whh_l0_d1, Whh_l1_d0, Whh_l1_d1,
        biases, fc_w.reshape(T * 5, H2))

    ids_tm = input_ids.T.reshape(M).astype(jnp.int32)
    x = _emb_gather(emb_table, ids_tm)

    hf0, hb0 = _fused_scan_l0(x.reshape(S, B, D), w0, b0, wf0, wb0)

    yf, yb = _fused_scan_l1(hf0, hb0, w1a, w1b, b1, wf1, wb1, fa, fb)
    y = yf[:, :, : T * 5] + yb[:, :, : T * 5] + fc_b.reshape(T * 5)
    logits = y.reshape(S, B, T, 5).transpose(1, 2, 0, 3)
    return logits


# speedup vs baseline: 1.0469x; 1.0251x over previous
import functools

import jax
import jax.numpy as jnp
from jax.experimental import pallas as pl
from jax.experimental.pallas import tpu as pltpu
from jax.experimental.pallas import tpu_sc as plsc

V, D, H2, L, T = 30000, 256, 512, 2, 4
H = H2 // 2
B, S = 32, 512
G4 = 4 * H
M = B * S

_WIN = 128


def _emb_gather(table, ids_flat):
    mesh = plsc.VectorSubcoreMesh(core_axis_name="core",
                                  subcore_axis_name="subcore")
    idx2 = ids_flat.reshape(1, M)

    @functools.partial(
        pl.kernel,
        out_type=jax.ShapeDtypeStruct((M, D), jnp.float32),
        mesh=mesh,
    )
    def k(tab_hbm, i_hbm, o_hbm):
        def body(i_vmem, o_vmem):
            pltpu.sync_copy(tab_hbm.at[i_vmem.at[0]], o_vmem)

        pltpu.emit_pipeline(
            body,
            grid=(M // _WIN,),
            in_specs=[pl.BlockSpec((1, _WIN), index_map=lambda i: (0, i))],
            out_specs=[pl.BlockSpec((_WIN, D), index_map=lambda i: (i, 0))],
            core_axis_name=("core", "subcore"),
            dimension_semantics=(pltpu.PARALLEL,),
        )(i_hbm, o_hbm)

    return k(table, idx2)


_CHUNK = 64
_NBLK = S // _CHUNK


def _lstm_step(x_gates, h, c, w):
    g = x_gates.astype(jnp.float32) + jnp.dot(
        h.astype(jnp.bfloat16), w, preferred_element_type=jnp.float32)
    sif = jax.nn.sigmoid(g[:, : 2 * H])
    gg = jnp.tanh(g[:, 2 * H: 3 * H])
    so = jax.nn.sigmoid(g[:, 3 * H:])
    c2 = sif[:, H:] * c + sif[:, :H] * gg
    h2 = so * jnp.tanh(c2)
    return h2, c2


def _zero_state(hf_ref, cf_ref, hb_ref, cb_ref):
    @pl.when(pl.program_id(0) == 0)
    def _():
        z = jnp.zeros((B, H), jnp.float32)
        hf_ref[...] = z
        cf_ref[...] = z
        hb_ref[...] = z
        cb_ref[...] = z


def _store_h(ref, j, h):
    if len(ref.shape) == 3:
        ref[j] = h.astype(jnp.bfloat16)
    else:
        ref[pl.ds(j * B, B)] = h.astype(jnp.bfloat16)


_NP = 1
_PP = _CHUNK // _NP


def _run_steps(projf, projb, xpf_refs, xpb_refs, wf_ref, wb_ref,
               of_ref, ob_ref, hf_ref, cf_ref, hb_ref, cb_ref):
    wf = wf_ref[...]
    wb = wb_ref[...]
    projf(0)
    projb(_NP - 1)
    for k in range(_NP):
        if k + 1 < _NP:
            projf(k + 1)
            projb(_NP - 2 - k)
        pb = _NP - 1 - k
        for j in range(k * _PP, (k + 1) * _PP):
            h2, c2 = _lstm_step(xpf_refs[k][pl.ds((j - k * _PP) * B, B)],
                                hf_ref[...], cf_ref[...], wf)
            hf_ref[...] = h2
            cf_ref[...] = c2
            _store_h(of_ref, j, h2)
            jb = _CHUNK - 1 - j
            h2, c2 = _lstm_step(xpb_refs[pb][pl.ds((jb - pb * _PP) * B, B)],
                                hb_ref[...], cb_ref[...], wb)
            hb_ref[...] = h2
            cb_ref[...] = c2
            _store_h(ob_ref, jb, h2)


def _prep_body(wi00_ref, wi01_ref, wi10_ref, wi11_ref,
               wh00_ref, wh01_ref, wh10_ref, wh11_ref,
               bi00_ref, bh00_ref, bi01_ref, bh01_ref,
               bi10_ref, bh10_ref, bi11_ref, bh11_ref, f2_ref,
               w0_ref, w1a_ref, w1b_ref, wf0_ref, wb0_ref, wf1_ref, wb1_ref,
               b0_ref, b1_ref, fa_ref, fb_ref):
    bf = jnp.bfloat16
    w0_ref[:, :G4] = wi00_ref[...].T.astype(bf)
    w0_ref[:, G4:] = wi01_ref[...].T.astype(bf)
    w1a_ref[:, :G4] = wi10_ref[:, :H].T.astype(bf)
    w1a_ref[:, G4:] = wi11_ref[:, :H].T.astype(bf)
    w1b_ref[:, :G4] = wi10_ref[:, H:].T.astype(bf)
    w1b_ref[:, G4:] = wi11_ref[:, H:].T.astype(bf)
    wf0_ref[...] = wh00_ref[...].T.astype(bf)
    wb0_ref[...] = wh01_ref[...].T.astype(bf)
    wf1_ref[...] = wh10_ref[...].T.astype(bf)
    wb1_ref[...] = wh11_ref[...].T.astype(bf)
    b0_ref[:, :G4] = bi00_ref[...] + bh00_ref[...]
    b0_ref[:, G4:] = bi01_ref[...] + bh01_ref[...]
    b1_ref[:, :G4] = bi10_ref[...] + bh10_ref[...]
    b1_ref[:, G4:] = bi11_ref[...] + bh11_ref[...]
    fa_ref[...] = jnp.zeros((H, 32), bf)
    fb_ref[...] = jnp.zeros((H, 32), bf)
    fa_ref[:, : T * 5] = f2_ref[:, :H].T.astype(bf)
    fb_ref[:, : T * 5] = f2_ref[:, H:].T.astype(bf)


def _prep_weights(wi00, wi01, wi10, wi11, wh00, wh01, wh10, wh11,
                  biases, f2):
    ins = [wi00, wi01, wi10, wi11, wh00, wh01, wh10, wh11] + biases + [f2]
    outs = [
        jax.ShapeDtypeStruct((D, 2 * G4), jnp.bfloat16),
        jax.ShapeDtypeStruct((H, 2 * G4), jnp.bfloat16),
        jax.ShapeDtypeStruct((H, 2 * G4), jnp.bfloat16),
        jax.ShapeDtypeStruct((H, G4), jnp.bfloat16),
        jax.ShapeDtypeStruct((H, G4), jnp.bfloat16),
        jax.ShapeDtypeStruct((H, G4), jnp.bfloat16),
        jax.ShapeDtypeStruct((H, G4), jnp.bfloat16),
        jax.ShapeDtypeStruct((1, 2 * G4), jnp.float32),
        jax.ShapeDtypeStruct((1, 2 * G4), jnp.float32),
        jax.ShapeDtypeStruct((H, 32), jnp.bfloat16),
        jax.ShapeDtypeStruct((H, 32), jnp.bfloat16),
    ]
    return pl.pallas_call(_prep_body, out_shape=outs)(*ins)


def _fscan1_body(xa_ref, xd_ref, win_ref, b_ref, wf_ref, wb_ref,
                 of_ref, ob_ref, *scratch):
    xpf_refs, xpb_refs = scratch[:_NP], scratch[_NP:2 * _NP]
    hf_ref, cf_ref, hb_ref, cb_ref = scratch[2 * _NP:]
    _zero_state(hf_ref, cf_ref, hb_ref, cb_ref)

    def projf(p):
        xa = xa_ref[pl.ds(p * _PP, _PP)].reshape(
            _PP * B, D).astype(jnp.bfloat16)
        xpf_refs[p][...] = jnp.dot(
            xa, win_ref[:, :G4],
            preferred_element_type=jnp.float32) + b_ref[:, :G4]

    def projb(p):
        xd = xd_ref[pl.ds(p * _PP, _PP)].reshape(
            _PP * B, D).astype(jnp.bfloat16)
        xpb_refs[p][...] = jnp.dot(
            xd, win_ref[:, G4:],
            preferred_element_type=jnp.float32) + b_ref[:, G4:]

    _run_steps(projf, projb, xpf_refs, xpb_refs, wf_ref, wb_ref,
               of_ref, ob_ref, hf_ref, cf_ref, hb_ref, cb_ref)


def _fused_scan_l0(x, win, b, wfT, wbT):
    return pl.pallas_call(
        _fscan1_body,
        grid=(_NBLK,),
        in_specs=[
            pl.BlockSpec((_CHUNK, B, D), lambda i: (i, 0, 0)),
            pl.BlockSpec((_CHUNK, B, D), lambda i: (_NBLK - 1 - i, 0, 0)),
            pl.BlockSpec((D, 2 * G4), lambda i: (0, 0)),
            pl.BlockSpec((1, 2 * G4), lambda i: (0, 0)),
            pl.BlockSpec((H, G4), lambda i: (0, 0)),
            pl.BlockSpec((H, G4), lambda i: (0, 0)),
        ],
        out_specs=[
            pl.BlockSpec((_CHUNK, B, H), lambda i: (i, 0, 0)),
            pl.BlockSpec((_CHUNK, B, H), lambda i: (_NBLK - 1 - i, 0, 0)),
        ],
        out_shape=[
            jax.ShapeDtypeStruct((S, B, H), jnp.bfloat16),
            jax.ShapeDtypeStruct((S, B, H), jnp.bfloat16),
        ],
        scratch_shapes=(
            [pltpu.VMEM((_PP * B, G4), jnp.float32) for _ in range(2 * _NP)]
            + [pltpu.VMEM((B, H), jnp.float32) for _ in range(4)]),
        compiler_params=pltpu.CompilerParams(
            dimension_semantics=("arbitrary",)),
    )(x, x, win, b.reshape(1, 2 * G4), wfT, wbT)


def _fscan2_body(ha_ref, hb0a_ref, hd_ref, hb0d_ref, wina_ref, winb_ref,
                 b_ref, wf_ref, wb_ref, fa_ref, fb_ref,
                 yf_ref, yb_ref, *scratch):
    xpf_refs, xpb_refs = scratch[:_NP], scratch[_NP:2 * _NP]
    osf_ref, osb_ref = scratch[2 * _NP:2 * _NP + 2]
    hf_ref, cf_ref, hb_ref, cb_ref = scratch[2 * _NP + 2:]
    _zero_state(hf_ref, cf_ref, hb_ref, cb_ref)

    def projf(p):
        ha = ha_ref[pl.ds(p * _PP, _PP)].reshape(_PP * B, H)
        h0a = hb0a_ref[pl.ds(p * _PP, _PP)].reshape(_PP * B, H)
        xpf_refs[p][...] = (
            jnp.dot(ha, wina_ref[:, :G4], preferred_element_type=jnp.float32)
            + jnp.dot(h0a, winb_ref[:, :G4],
                      preferred_element_type=jnp.float32)
            + b_ref[:, :G4])

    def projb(p):
        hd = hd_ref[pl.ds(p * _PP, _PP)].reshape(_PP * B, H)
        h0d = hb0d_ref[pl.ds(p * _PP, _PP)].reshape(_PP * B, H)
        xpb_refs[p][...] = (
            jnp.dot(hd, wina_ref[:, G4:], preferred_element_type=jnp.float32)
            + jnp.dot(h0d, winb_ref[:, G4:],
                      preferred_element_type=jnp.float32)
            + b_ref[:, G4:])

    _run_steps(projf, projb, xpf_refs, xpb_refs, wf_ref, wb_ref,
               osf_ref, osb_ref, hf_ref, cf_ref, hb_ref, cb_ref)
    yf_ref[...] = jnp.dot(
        osf_ref[...], fa_ref[...],
        preferred_element_type=jnp.float32).reshape(_CHUNK, B, 32)
    yb_ref[...] = jnp.dot(
        osb_ref[...], fb_ref[...],
        preferred_element_type=jnp.float32).reshape(_CHUNK, B, 32)


def _fused_scan_l1(hf0, hb0, wina, winb, b, wfT, wbT, fa, fb):
    return pl.pallas_call(
        _fscan2_body,
        grid=(_NBLK,),
        in_specs=[
            pl.BlockSpec((_CHUNK, B, H), lambda i: (i, 0, 0)),
            pl.BlockSpec((_CHUNK, B, H), lambda i: (i, 0, 0)),
            pl.BlockSpec((_CHUNK, B, H), lambda i: (_NBLK - 1 - i, 0, 0)),
            pl.BlockSpec((_CHUNK, B, H), lambda i: (_NBLK - 1 - i, 0, 0)),
            pl.BlockSpec((H, 2 * G4), lambda i: (0, 0)),
            pl.BlockSpec((H, 2 * G4), lambda i: (0, 0)),
            pl.BlockSpec((1, 2 * G4), lambda i: (0, 0)),
            pl.BlockSpec((H, G4), lambda i: (0, 0)),
            pl.BlockSpec((H, G4), lambda i: (0, 0)),
            pl.BlockSpec((H, 32), lambda i: (0, 0)),
            pl.BlockSpec((H, 32), lambda i: (0, 0)),
        ],
        out_specs=[
            pl.BlockSpec((_CHUNK, B, 32), lambda i: (i, 0, 0)),
            pl.BlockSpec((_CHUNK, B, 32), lambda i: (_NBLK - 1 - i, 0, 0)),
        ],
        out_shape=[
            jax.ShapeDtypeStruct((S, B, 32), jnp.float32),
            jax.ShapeDtypeStruct((S, B, 32), jnp.float32),
        ],
        scratch_shapes=(
            [pltpu.VMEM((_PP * B, G4), jnp.float32) for _ in range(2 * _NP)]
            + [pltpu.VMEM((_CHUNK * B, H), jnp.bfloat16) for _ in range(2)]
            + [pltpu.VMEM((B, H), jnp.float32) for _ in range(4)]),
        compiler_params=pltpu.CompilerParams(
            dimension_semantics=("arbitrary",)),
    )(hf0, hb0, hf0, hb0, wina, winb, b.reshape(1, 2 * G4), wfT, wbT, fa, fb)


def kernel(input_ids, emb_table, Wih_l0_d0, Whh_l0_d0, bih_l0_d0, bhh_l0_d0,
           Wih_l0_d1, Whh_l0_d1, bih_l0_d1, bhh_l0_d1, Wih_l1_d0, Whh_l1_d0,
           bih_l1_d0, bhh_l1_d0, Wih_l1_d1, Whh_l1_d1, bih_l1_d1, bhh_l1_d1,
           fc_w, fc_b):
    biases = [b.reshape(1, G4) for b in
              (bih_l0_d0, bhh_l0_d0, bih_l0_d1, bhh_l0_d1,
               bih_l1_d0, bhh_l1_d0, bih_l1_d1, bhh_l1_d1)]
    (w0, w1a, w1b, wf0, wb0, wf1, wb1, b0, b1, fa, fb) = _prep_weights(
        Wih_l0_d0, Wih_l0_d1, Wih_l1_d0, Wih_l1_d1,
        Whh_l0_d0, Whh_l0_d1, Whh_l1_d0, Whh_l1_d1,
        biases, fc_w.reshape(T * 5, H2))

    ids_tm = input_ids.T.reshape(M).astype(jnp.int32)
    x = _emb_gather(emb_table, ids_tm)

    hf0, hb0 = _fused_scan_l0(x.reshape(S, B, D), w0, b0, wf0, wb0)

    yf, yb = _fused_scan_l1(hf0, hb0, w1a, w1b, b1, wf1, wb1, fa, fb)
    y = yf[:, :, : T * 5] + yb[:, :, : T * 5] + fc_b.reshape(T * 5)
    logits = y.reshape(S, B, T, 5).transpose(1, 2, 0, 3)
    return logits
